# baseline (speedup 1.0000x reference)
import functools

import jax
import jax.numpy as jnp
from jax import lax
from jax.experimental import pallas as pl
from jax.experimental.pallas import tpu as pltpu
from jax.experimental.pallas import tpu_sc as plsc

N_NODES = 10000
CHUNK = 128
NC = 2
NS = 16
N_PAD = 10112


def _seg_sum_kernel(nct0, nct1, width):
    assert nct0 % 2 == 0 and nct1 % 2 == 0
    k0 = NS * nct0
    mesh = plsc.VectorSubcoreMesh(core_axis_name="c", subcore_axis_name="s")
    rows_per_tile = N_PAD // NS

    @functools.partial(
        pl.kernel,
        out_type=jax.ShapeDtypeStruct((NC, N_PAD, width), jnp.float32),
        mesh=mesh,
        compiler_params=pltpu.CompilerParams(
            use_tc_tiling_on_sc=(width % 128 == 0)),
        scratch_types=[
            pltpu.VMEM((CHUNK,), jnp.int32),
            pltpu.VMEM((CHUNK,), jnp.int32),
            pltpu.VMEM((CHUNK,), jnp.int32),
            pltpu.VMEM((CHUNK,), jnp.int32),
            pltpu.VMEM((CHUNK, width), jnp.float32),
            pltpu.VMEM((CHUNK, width), jnp.float32),
            pltpu.VMEM_SHARED((N_PAD, width), jnp.float32),
            pltpu.SemaphoreType.DMA,
            pltpu.SemaphoreType.DMA,
            pltpu.SemaphoreType.DMA,
            pltpu.SemaphoreType.DMA,
            pltpu.SemaphoreType.DMA,
            pltpu.SemaphoreType.DMA,
            pltpu.SemaphoreType.DMA,
            pltpu.SemaphoreType.DMA,
            pltpu.SemaphoreType.DMA,
        ],
    )
    def seg_sum(sup_hbm, src_hbm, dst_hbm, out_hbm,
                sidx0, sidx1, didx0, didx1, rows0, rows1, acc_sh,
                isem0, isem1, dsem0, dsem1, gsem0, gsem1, ssem0, ssem1, zsem):
        cid = lax.axis_index("c")
        sid = lax.axis_index("s")

        zrow = sid * rows_per_tile
        zv = jnp.zeros((16,), jnp.float32)

        def zbody(r, carry):
            for c in range(width // 16):
                rows0[r, pl.ds(c * 16, 16)] = zv
            return carry

        lax.fori_loop(0, CHUNK, zbody, 0)
        nfull = rows_per_tile // CHUNK
        rem = rows_per_tile - nfull * CHUNK
        for kz in range(nfull):
            pltpu.async_copy(rows0, acc_sh.at[pl.ds(zrow + kz * CHUNK, CHUNK)],
                             zsem)
        pltpu.async_copy(rows0.at[pl.ds(0, rem)],
                         acc_sh.at[pl.ds(zrow + nfull * CHUNK, rem)], zsem)
        for kz in range(nfull):
            pltpu.make_async_copy(
                rows0, acc_sh.at[pl.ds(zrow + kz * CHUNK, CHUNK)], zsem).wait()
        pltpu.make_async_copy(
            rows0.at[pl.ds(0, rem)],
            acc_sh.at[pl.ds(zrow + nfull * CHUNK, rem)], zsem).wait()

        def i_start(k, buf, sem):
            pltpu.async_copy(src_hbm.at[k], buf, sem)

        def i_wait(k, buf, sem):
            pltpu.make_async_copy(src_hbm.at[k], buf, sem).wait()

        def d_start(k, buf, sem):
            pltpu.async_copy(dst_hbm.at[k], buf, sem)

        def d_wait(k, buf, sem):
            pltpu.make_async_copy(dst_hbm.at[k], buf, sem).wait()

        def g_start(buf, ibuf, sem):
            pltpu.async_copy(sup_hbm.at[ibuf], buf, sem)

        def g_wait(buf, ibuf, sem):
            pltpu.make_async_copy(sup_hbm.at[ibuf], buf, sem).wait()

        def s_start(buf, dbuf, sem):
            pltpu.async_copy(buf, acc_sh.at[dbuf], sem, add=True)

        def s_wait(buf, dbuf, sem):
            pltpu.make_async_copy(buf, acc_sh.at[dbuf], sem).wait()

        def run(nct, base_k):
            i_start(base_k, sidx0, isem0)
            i_start(base_k + 1, sidx1, isem1)
            d_start(base_k, didx0, dsem0)
            d_start(base_k + 1, didx1, dsem1)
            plsc.subcore_barrier()
            i_wait(base_k, sidx0, isem0)
            g_start(rows0, sidx0, gsem0)
            i_wait(base_k + 1, sidx1, isem1)
            g_start(rows1, sidx1, gsem1)

            def body(p, carry):
                i0 = 2 * p
                g_wait(rows0, sidx0, gsem0)
                i_start(base_k + i0 + 2, sidx0, isem0)
                d_wait(base_k + i0, didx0, dsem0)
                s_start(rows0, didx0, ssem0)
                g_wait(rows1, sidx1, gsem1)
                i_start(base_k + i0 + 3, sidx1, isem1)
                d_wait(base_k + i0 + 1, didx1, dsem1)
                s_start(rows1, didx1, ssem1)
                s_wait(rows0, didx0, ssem0)
                d_start(base_k + i0 + 2, didx0, dsem0)
                i_wait(base_k + i0 + 2, sidx0, isem0)
                g_start(rows0, sidx0, gsem0)
                s_wait(rows1, didx1, ssem1)
                d_start(base_k + i0 + 3, didx1, dsem1)
                i_wait(base_k + i0 + 3, sidx1, isem1)
                g_start(rows1, sidx1, gsem1)
                return carry

            lax.fori_loop(0, nct // 2 - 1, body, 0)

            last = base_k + nct - 2
            g_wait(rows0, sidx0, gsem0)
            d_wait(last, didx0, dsem0)
            s_start(rows0, didx0, ssem0)
            g_wait(rows1, sidx1, gsem1)
            d_wait(last + 1, didx1, dsem1)
            s_start(rows1, didx1, ssem1)
            s_wait(rows0, didx0, ssem0)
            s_wait(rows1, didx1, ssem1)
            plsc.subcore_barrier()

        @pl.when(cid == 0)
        def _():
            run(nct0, sid * nct0)

        @pl.when(cid == 1)
        def _():
            run(nct1, k0 + sid * nct1)

        pltpu.sync_copy(acc_sh.at[pl.ds(zrow, rows_per_tile)],
                        out_hbm.at[cid, pl.ds(zrow, rows_per_tile)])

    return seg_sum


def _stage_a1_body(x_ref, w1_ref, s1_ref):
    s1_ref[...] = jnp.dot(x_ref[...], w1_ref[...],
                          preferred_element_type=jnp.float32)


def _stage_a2_body(x_ref, dsw_ref, dsb_ref, ox_ref):
    ox_ref[...] = jnp.dot(x_ref[...], dsw_ref[...],
                          preferred_element_type=jnp.float32) + dsb_ref[...]


def _stage_b1_body(pa_ref, pb_ref, b1_ref, w3_ref, s3_ref):
    h = jnp.maximum(pa_ref[...] + pb_ref[...] + b1_ref[...], 0.0)
    s3_ref[...] = jnp.dot(h, w3_ref[...], preferred_element_type=jnp.float32)


def _stage_b2_body(pa_ref, pb_ref, b1_ref, w2_ref, s2_ref):
    h = jnp.maximum(pa_ref[...] + pb_ref[...] + b1_ref[...], 0.0)
    s2_ref[...] = jnp.dot(h, w2_ref[...], preferred_element_type=jnp.float32)


def _stage_c1_body(p3a_ref, p3b_ref, ox_ref, b3_ref,
                   l2w_ref, l2b_ref, l3w_ref, l3b_ref, r2_ref, r3_ref):
    h2 = jnp.maximum(p3a_ref[...] + p3b_ref[...] + b3_ref[...], 0.0)
    h2 = h2 + ox_ref[...]
    r2_ref[...] = jnp.dot(h2, l2w_ref[...],
                          preferred_element_type=jnp.float32) + l2b_ref[...]
    r3_ref[...] = jnp.dot(h2, l3w_ref[...],
                          preferred_element_type=jnp.float32) + l3b_ref[...]


def _stage_c2_body(p2a_ref, p2b_ref, b2_ref, r1_ref):
    nclass = b2_ref.shape[1]
    c = (p2a_ref[...] + p2b_ref[...])[:, :nclass] + b2_ref[...]
    c = c - jnp.max(c, axis=1, keepdims=True)
    r1_ref[...] = c - jnp.log(jnp.sum(jnp.exp(c), axis=1, keepdims=True))


_ROW_BLK = N_PAD // 8


def _row_spec(cols):
    return pl.BlockSpec((_ROW_BLK, cols), lambda i: (i, 0))


def _full_spec(rows, cols):
    return pl.BlockSpec((rows, cols), lambda i: (0, 0))


def kernel(x, adj, gc1_W, gc1_b, gc2_W, gc2_b, gc3_W, gc3_b, ds_W, ds_b,
           lin2_W, lin2_b, lin3_W, lin3_b):
    n, d = x.shape
    e = adj.shape[1]
    nclass = gc2_W.shape[1]
    ndeg = lin3_W.shape[1]
    w2pad = 48

    epw = NC * NS * CHUNK * 2
    e_pad = ((e + epw - 1) // epw) * epw
    pad = e_pad - e
    total_chunks_per_tile = e_pad // (NC * NS * CHUNK)
    nct0 = total_chunks_per_tile + 72
    nct1 = 2 * total_chunks_per_tile - nct0
    src = jnp.concatenate([adj[0], jnp.zeros((pad,), jnp.int32)])
    dst = jnp.concatenate([adj[1], jnp.full((pad,), n, jnp.int32)])
    src = src.reshape(e_pad // CHUNK, CHUNK)
    dst = dst.reshape(e_pad // CHUNK, CHUNK)
    x_p = jnp.concatenate([x, jnp.zeros((N_PAD - n, d), jnp.float32)])
    gc2_Wp = jnp.concatenate(
        [gc2_W, jnp.zeros((d, w2pad - nclass), jnp.float32)], axis=1)

    seg_sum_d = _seg_sum_kernel(nct0, nct1, d)
    seg_sum_n = _seg_sum_kernel(nct0, nct1, w2pad)

    s1 = pl.pallas_call(
        _stage_a1_body,
        grid=(N_PAD // _ROW_BLK,),
        in_specs=[_row_spec(d), _full_spec(d, d)],
        out_specs=_row_spec(d),
        out_shape=jax.ShapeDtypeStruct((N_PAD, d), jnp.float32),
    )(x_p, gc1_W)

    p1 = seg_sum_d(s1, src, dst)

    ox = pl.pallas_call(
        _stage_a2_body,
        grid=(N_PAD // _ROW_BLK,),
        in_specs=[_row_spec(d), _full_spec(d, d), _full_spec(1, d)],
        out_specs=_row_spec(d),
        out_shape=jax.ShapeDtypeStruct((N_PAD, d), jnp.float32),
    )(x_p, ds_W, ds_b.reshape(1, d))

    s3 = pl.pallas_call(
        _stage_b1_body,
        grid=(N_PAD // _ROW_BLK,),
        in_specs=[_row_spec(d), _row_spec(d), _full_spec(1, d),
                  _full_spec(d, d)],
        out_specs=_row_spec(d),
        out_shape=jax.ShapeDtypeStruct((N_PAD, d), jnp.float32),
    )(p1[0], p1[1], gc1_b.reshape(1, d), gc3_W)

    p3 = seg_sum_d(s3, src, dst)

    s2 = pl.pallas_call(
        _stage_b2_body,
        grid=(N_PAD // _ROW_BLK,),
        in_specs=[_row_spec(d), _row_spec(d), _full_spec(1, d),
                  _full_spec(d, w2pad)],
        out_specs=_row_spec(w2pad),
        out_shape=jax.ShapeDtypeStruct((N_PAD, w2pad), jnp.float32),
    )(p1[0], p1[1], gc1_b.reshape(1, d), gc2_Wp)

    p2 = seg_sum_n(s2, src, dst)

    r2, r3 = pl.pallas_call(
        _stage_c1_body,
        grid=(N_PAD // _ROW_BLK,),
        in_specs=[
            _row_spec(d), _row_spec(d), _row_spec(d), _full_spec(1, d),
            _full_spec(d, 1), _full_spec(1, 1),
            _full_spec(d, ndeg), _full_spec(1, ndeg),
        ],
        out_specs=[_row_spec(1), _row_spec(ndeg)],
        out_shape=[
            jax.ShapeDtypeStruct((N_PAD, 1), jnp.float32),
            jax.ShapeDtypeStruct((N_PAD, ndeg), jnp.float32),
        ],
    )(p3[0], p3[1], ox, gc3_b.reshape(1, d),
      lin2_W, lin2_b.reshape(1, 1),
      lin3_W, lin3_b.reshape(1, ndeg))

    r1 = pl.pallas_call(
        _stage_c2_body,
        grid=(N_PAD // _ROW_BLK,),
        in_specs=[_row_spec(w2pad), _row_spec(w2pad), _full_spec(1, nclass)],
        out_specs=_row_spec(nclass),
        out_shape=jax.ShapeDtypeStruct((N_PAD, nclass), jnp.float32),
    )(p2[0], p2[1], gc2_b.reshape(1, nclass))

    return (r1[:n], r2[:n, 0], r3[:n])


# speedup vs baseline: 1.0177x; 1.0177x over previous
import functools

import jax
import jax.numpy as jnp
from jax import lax
from jax.experimental import pallas as pl
from jax.experimental.pallas import tpu as pltpu
from jax.experimental.pallas import tpu_sc as plsc

N_NODES = 10000
CHUNK = 128
NC = 2
NS = 16
N_PAD = 10112


def _seg_sum_kernel(nct0, nct1, width):
    assert nct0 % 2 == 0 and nct1 % 2 == 0
    k0 = NS * nct0
    mesh = plsc.VectorSubcoreMesh(core_axis_name="c", subcore_axis_name="s")
    rows_per_tile = N_PAD // NS

    @functools.partial(
        pl.kernel,
        out_type=jax.ShapeDtypeStruct((NC, N_PAD, width), jnp.float32),
        mesh=mesh,
        compiler_params=pltpu.CompilerParams(
            use_tc_tiling_on_sc=(width % 128 == 0)),
        scratch_types=[
            pltpu.VMEM((CHUNK,), jnp.int32),
            pltpu.VMEM((CHUNK,), jnp.int32),
            pltpu.VMEM((CHUNK,), jnp.int32),
            pltpu.VMEM((CHUNK,), jnp.int32),
            pltpu.VMEM((CHUNK, width), jnp.float32),
            pltpu.VMEM((CHUNK, width), jnp.float32),
            pltpu.VMEM_SHARED((N_PAD, width), jnp.float32),
            pltpu.SemaphoreType.DMA,
            pltpu.SemaphoreType.DMA,
            pltpu.SemaphoreType.DMA,
            pltpu.SemaphoreType.DMA,
            pltpu.SemaphoreType.DMA,
            pltpu.SemaphoreType.DMA,
            pltpu.SemaphoreType.DMA,
            pltpu.SemaphoreType.DMA,
            pltpu.SemaphoreType.DMA,
        ],
    )
    def seg_sum(sup_hbm, src_hbm, dst_hbm, out_hbm,
                sidx0, sidx1, didx0, didx1, rows0, rows1, acc_sh,
                isem0, isem1, dsem0, dsem1, gsem0, gsem1, ssem0, ssem1, zsem):
        cid = lax.axis_index("c")
        sid = lax.axis_index("s")

        zrow = sid * rows_per_tile
        zv = jnp.zeros((16,), jnp.float32)

        def zbody(r, carry):
            for c in range(width // 16):
                rows0[r, pl.ds(c * 16, 16)] = zv
            return carry

        lax.fori_loop(0, CHUNK, zbody, 0)
        nfull = rows_per_tile // CHUNK
        rem = rows_per_tile - nfull * CHUNK
        for kz in range(nfull):
            pltpu.async_copy(rows0, acc_sh.at[pl.ds(zrow + kz * CHUNK, CHUNK)],
                             zsem)
        pltpu.async_copy(rows0.at[pl.ds(0, rem)],
                         acc_sh.at[pl.ds(zrow + nfull * CHUNK, rem)], zsem)
        for kz in range(nfull):
            pltpu.make_async_copy(
                rows0, acc_sh.at[pl.ds(zrow + kz * CHUNK, CHUNK)], zsem).wait()
        pltpu.make_async_copy(
            rows0.at[pl.ds(0, rem)],
            acc_sh.at[pl.ds(zrow + nfull * CHUNK, rem)], zsem).wait()

        def i_start(k, buf, sem):
            pltpu.async_copy(src_hbm.at[k], buf, sem)

        def i_wait(k, buf, sem):
            pltpu.make_async_copy(src_hbm.at[k], buf, sem).wait()

        def d_start(k, buf, sem):
            pltpu.async_copy(dst_hbm.at[k], buf, sem)

        def d_wait(k, buf, sem):
            pltpu.make_async_copy(dst_hbm.at[k], buf, sem).wait()

        def g_start(buf, ibuf, sem):
            pltpu.async_copy(sup_hbm.at[ibuf], buf, sem)

        def g_wait(buf, ibuf, sem):
            pltpu.make_async_copy(sup_hbm.at[ibuf], buf, sem).wait()

        def s_start(buf, dbuf, sem):
            pltpu.async_copy(buf, acc_sh.at[dbuf], sem, add=True)

        def s_wait(buf, dbuf, sem):
            pltpu.make_async_copy(buf, acc_sh.at[dbuf], sem).wait()

        def run(nct, base_k):
            i_start(base_k, sidx0, isem0)
            i_start(base_k + 1, sidx1, isem1)
            d_start(base_k, didx0, dsem0)
            d_start(base_k + 1, didx1, dsem1)
            plsc.subcore_barrier()
            i_wait(base_k, sidx0, isem0)
            g_start(rows0, sidx0, gsem0)
            i_wait(base_k + 1, sidx1, isem1)
            g_start(rows1, sidx1, gsem1)

            def body(p, carry):
                i0 = 2 * p
                g_wait(rows0, sidx0, gsem0)
                i_start(base_k + i0 + 2, sidx0, isem0)
                d_wait(base_k + i0, didx0, dsem0)
                s_start(rows0, didx0, ssem0)
                g_wait(rows1, sidx1, gsem1)
                i_start(base_k + i0 + 3, sidx1, isem1)
                d_wait(base_k + i0 + 1, didx1, dsem1)
                s_start(rows1, didx1, ssem1)
                s_wait(rows0, didx0, ssem0)
                d_start(base_k + i0 + 2, didx0, dsem0)
                i_wait(base_k + i0 + 2, sidx0, isem0)
                g_start(rows0, sidx0, gsem0)
                s_wait(rows1, didx1, ssem1)
                d_start(base_k + i0 + 3, didx1, dsem1)
                i_wait(base_k + i0 + 3, sidx1, isem1)
                g_start(rows1, sidx1, gsem1)
                return carry

            lax.fori_loop(0, nct // 2 - 1, body, 0)

            last = base_k + nct - 2
            g_wait(rows0, sidx0, gsem0)
            d_wait(last, didx0, dsem0)
            s_start(rows0, didx0, ssem0)
            g_wait(rows1, sidx1, gsem1)
            d_wait(last + 1, didx1, dsem1)
            s_start(rows1, didx1, ssem1)
            s_wait(rows0, didx0, ssem0)
            s_wait(rows1, didx1, ssem1)
            plsc.subcore_barrier()

        @pl.when(cid == 0)
        def _():
            run(nct0, sid * nct0)

        @pl.when(cid == 1)
        def _():
            run(nct1, k0 + sid * nct1)

        pltpu.sync_copy(acc_sh.at[pl.ds(zrow, rows_per_tile)],
                        out_hbm.at[cid, pl.ds(zrow, rows_per_tile)])

    return seg_sum


def _stage_a_body(x_ref, w1_ref, dsw_ref, dsb_ref, s1_ref, ox_ref):
    x = x_ref[...]
    s1_ref[...] = jnp.dot(x, w1_ref[...], preferred_element_type=jnp.float32)
    ox_ref[...] = jnp.dot(x, dsw_ref[...],
                          preferred_element_type=jnp.float32) + dsb_ref[...]


def _stage_b_body(pa_ref, pb_ref, b1_ref, w2_ref, w3_ref, s2_ref, s3_ref):
    h = jnp.maximum(pa_ref[...] + pb_ref[...] + b1_ref[...], 0.0)
    s2_ref[...] = jnp.dot(h, w2_ref[...], preferred_element_type=jnp.float32)
    s3_ref[...] = jnp.dot(h, w3_ref[...], preferred_element_type=jnp.float32)


def _stage_c1_body(p3a_ref, p3b_ref, ox_ref, b3_ref,
                   l2w_ref, l2b_ref, l3w_ref, l3b_ref, r2_ref, r3_ref):
    h2 = jnp.maximum(p3a_ref[...] + p3b_ref[...] + b3_ref[...], 0.0)
    h2 = h2 + ox_ref[...]
    r2_ref[...] = jnp.dot(h2, l2w_ref[...],
                          preferred_element_type=jnp.float32) + l2b_ref[...]
    r3_ref[...] = jnp.dot(h2, l3w_ref[...],
                          preferred_element_type=jnp.float32) + l3b_ref[...]


def _stage_c2_body(p2a_ref, p2b_ref, b2_ref, r1_ref):
    nclass = b2_ref.shape[1]
    c = (p2a_ref[...] + p2b_ref[...])[:, :nclass] + b2_ref[...]
    c = c - jnp.max(c, axis=1, keepdims=True)
    r1_ref[...] = c - jnp.log(jnp.sum(jnp.exp(c), axis=1, keepdims=True))


_ROW_BLK = N_PAD // 8


def _row_spec(cols):
    return pl.BlockSpec((_ROW_BLK, cols), lambda i: (i, 0))


def _full_spec(rows, cols):
    return pl.BlockSpec((rows, cols), lambda i: (0, 0))


def kernel(x, adj, gc1_W, gc1_b, gc2_W, gc2_b, gc3_W, gc3_b, ds_W, ds_b,
           lin2_W, lin2_b, lin3_W, lin3_b):
    n, d = x.shape
    e = adj.shape[1]
    nclass = gc2_W.shape[1]
    ndeg = lin3_W.shape[1]
    w2pad = 48

    epw = NC * NS * CHUNK * 2
    e_pad = ((e + epw - 1) // epw) * epw
    pad = e_pad - e
    total_chunks_per_tile = e_pad // (NC * NS * CHUNK)
    nct0 = total_chunks_per_tile + 72
    nct1 = 2 * total_chunks_per_tile - nct0
    src = jnp.concatenate([adj[0], jnp.zeros((pad,), jnp.int32)])
    dst = jnp.concatenate([adj[1], jnp.full((pad,), n, jnp.int32)])
    src = src.reshape(e_pad // CHUNK, CHUNK)
    dst = dst.reshape(e_pad // CHUNK, CHUNK)
    x_p = jnp.concatenate([x, jnp.zeros((N_PAD - n, d), jnp.float32)])
    gc2_Wp = jnp.concatenate(
        [gc2_W, jnp.zeros((d, w2pad - nclass), jnp.float32)], axis=1)

    seg_sum_d = _seg_sum_kernel(nct0, nct1, d)
    seg_sum_n = _seg_sum_kernel(nct0, nct1, w2pad)

    s1, ox = pl.pallas_call(
        _stage_a_body,
        grid=(N_PAD // _ROW_BLK,),
        in_specs=[_row_spec(d), _full_spec(d, d), _full_spec(d, d),
                  _full_spec(1, d)],
        out_specs=[_row_spec(d), _row_spec(d)],
        out_shape=[jax.ShapeDtypeStruct((N_PAD, d), jnp.float32),
                   jax.ShapeDtypeStruct((N_PAD, d), jnp.float32)],
    )(x_p, gc1_W, ds_W, ds_b.reshape(1, d))

    p1 = seg_sum_d(s1, src, dst)

    s2, s3 = pl.pallas_call(
        _stage_b_body,
        grid=(N_PAD // _ROW_BLK,),
        in_specs=[_row_spec(d), _row_spec(d), _full_spec(1, d),
                  _full_spec(d, w2pad), _full_spec(d, d)],
        out_specs=[_row_spec(w2pad), _row_spec(d)],
        out_shape=[jax.ShapeDtypeStruct((N_PAD, w2pad), jnp.float32),
                   jax.ShapeDtypeStruct((N_PAD, d), jnp.float32)],
    )(p1[0], p1[1], gc1_b.reshape(1, d), gc2_Wp, gc3_W)

    p3 = seg_sum_d(s3, src, dst)
    p2 = seg_sum_n(s2, src, dst)

    r2, r3 = pl.pallas_call(
        _stage_c1_body,
        grid=(N_PAD // _ROW_BLK,),
        in_specs=[
            _row_spec(d), _row_spec(d), _row_spec(d), _full_spec(1, d),
            _full_spec(d, 1), _full_spec(1, 1),
            _full_spec(d, ndeg), _full_spec(1, ndeg),
        ],
        out_specs=[_row_spec(1), _row_spec(ndeg)],
        out_shape=[
            jax.ShapeDtypeStruct((N_PAD, 1), jnp.float32),
            jax.ShapeDtypeStruct((N_PAD, ndeg), jnp.float32),
        ],
    )(p3[0], p3[1], ox, gc3_b.reshape(1, d),
      lin2_W, lin2_b.reshape(1, 1),
      lin3_W, lin3_b.reshape(1, ndeg))

    r1 = pl.pallas_call(
        _stage_c2_body,
        grid=(N_PAD // _ROW_BLK,),
        in_specs=[_row_spec(w2pad), _row_spec(w2pad), _full_spec(1, nclass)],
        out_specs=_row_spec(nclass),
        out_shape=jax.ShapeDtypeStruct((N_PAD, nclass), jnp.float32),
    )(p2[0], p2[1], gc2_b.reshape(1, nclass))

    return (r1[:n], r2[:n, 0], r3[:n])


# speedup vs baseline: 1.0314x; 1.0134x over previous
import functools

import jax
import jax.numpy as jnp
from jax import lax
from jax.experimental import pallas as pl
from jax.experimental.pallas import tpu as pltpu
from jax.experimental.pallas import tpu_sc as plsc

N_NODES = 10000
CHUNK = 128
NC = 2
NS = 16
N_PAD = 10112


def _seg_sum_kernel(nct0, nct1, width):
    assert nct0 % 2 == 0 and nct1 % 2 == 0
    k0 = NS * nct0
    mesh = plsc.VectorSubcoreMesh(core_axis_name="c", subcore_axis_name="s")
    rows_per_tile = N_PAD // NS

    @functools.partial(
        pl.kernel,
        out_type=jax.ShapeDtypeStruct((NC, N_PAD, width), jnp.float32),
        mesh=mesh,
        compiler_params=pltpu.CompilerParams(
            use_tc_tiling_on_sc=(width % 128 == 0)),
        scratch_types=[
            pltpu.VMEM((CHUNK,), jnp.int32),
            pltpu.VMEM((CHUNK,), jnp.int32),
            pltpu.VMEM((CHUNK,), jnp.int32),
            pltpu.VMEM((CHUNK,), jnp.int32),
            pltpu.VMEM((CHUNK, width), jnp.float32),
            pltpu.VMEM((CHUNK, width), jnp.float32),
            pltpu.VMEM_SHARED((N_PAD, width), jnp.float32),
            pltpu.SemaphoreType.DMA,
            pltpu.SemaphoreType.DMA,
            pltpu.SemaphoreType.DMA,
            pltpu.SemaphoreType.DMA,
            pltpu.SemaphoreType.DMA,
            pltpu.SemaphoreType.DMA,
            pltpu.SemaphoreType.DMA,
            pltpu.SemaphoreType.DMA,
            pltpu.SemaphoreType.DMA,
        ],
    )
    def seg_sum(sup_hbm, src_hbm, dst_hbm, out_hbm,
                sidx0, sidx1, didx0, didx1, rows0, rows1, acc_sh,
                isem0, isem1, dsem0, dsem1, gsem0, gsem1, ssem0, ssem1, zsem):
        cid = lax.axis_index("c")
        sid = lax.axis_index("s")

        zrow = sid * rows_per_tile
        zv = jnp.zeros((16,), jnp.float32)

        def zbody(r, carry):
            for c in range(width // 16):
                rows0[r, pl.ds(c * 16, 16)] = zv
            return carry

        lax.fori_loop(0, CHUNK, zbody, 0)
        nfull = rows_per_tile // CHUNK
        rem = rows_per_tile - nfull * CHUNK
        for kz in range(nfull):
            pltpu.async_copy(rows0, acc_sh.at[pl.ds(zrow + kz * CHUNK, CHUNK)],
                             zsem)
        pltpu.async_copy(rows0.at[pl.ds(0, rem)],
                         acc_sh.at[pl.ds(zrow + nfull * CHUNK, rem)], zsem)
        for kz in range(nfull):
            pltpu.make_async_copy(
                rows0, acc_sh.at[pl.ds(zrow + kz * CHUNK, CHUNK)], zsem).wait()
        pltpu.make_async_copy(
            rows0.at[pl.ds(0, rem)],
            acc_sh.at[pl.ds(zrow + nfull * CHUNK, rem)], zsem).wait()

        def i_start(k, buf, sem):
            pltpu.async_copy(src_hbm.at[k], buf, sem)

        def i_wait(k, buf, sem):
            pltpu.make_async_copy(src_hbm.at[k], buf, sem).wait()

        def d_start(k, buf, sem):
            pltpu.async_copy(dst_hbm.at[k], buf, sem)

        def d_wait(k, buf, sem):
            pltpu.make_async_copy(dst_hbm.at[k], buf, sem).wait()

        def g_start(buf, ibuf, sem):
            pltpu.async_copy(sup_hbm.at[ibuf], buf, sem)

        def g_wait(buf, ibuf, sem):
            pltpu.make_async_copy(sup_hbm.at[ibuf], buf, sem).wait()

        def s_start(buf, dbuf, sem):
            pltpu.async_copy(buf, acc_sh.at[dbuf], sem, add=True)

        def s_wait(buf, dbuf, sem):
            pltpu.make_async_copy(buf, acc_sh.at[dbuf], sem).wait()

        def run(nct, base_k):
            i_start(base_k, sidx0, isem0)
            i_start(base_k + 1, sidx1, isem1)
            d_start(base_k, didx0, dsem0)
            d_start(base_k + 1, didx1, dsem1)
            plsc.subcore_barrier()
            i_wait(base_k, sidx0, isem0)
            g_start(rows0, sidx0, gsem0)
            i_wait(base_k + 1, sidx1, isem1)
            g_start(rows1, sidx1, gsem1)

            def body(p, carry):
                i0 = 2 * p
                g_wait(rows0, sidx0, gsem0)
                i_start(base_k + i0 + 2, sidx0, isem0)
                d_wait(base_k + i0, didx0, dsem0)
                s_start(rows0, didx0, ssem0)
                g_wait(rows1, sidx1, gsem1)
                i_start(base_k + i0 + 3, sidx1, isem1)
                d_wait(base_k + i0 + 1, didx1, dsem1)
                s_start(rows1, didx1, ssem1)
                s_wait(rows0, didx0, ssem0)
                d_start(base_k + i0 + 2, didx0, dsem0)
                i_wait(base_k + i0 + 2, sidx0, isem0)
                g_start(rows0, sidx0, gsem0)
                s_wait(rows1, didx1, ssem1)
                d_start(base_k + i0 + 3, didx1, dsem1)
                i_wait(base_k + i0 + 3, sidx1, isem1)
                g_start(rows1, sidx1, gsem1)
                return carry

            lax.fori_loop(0, nct // 2 - 1, body, 0)

            last = base_k + nct - 2
            g_wait(rows0, sidx0, gsem0)
            d_wait(last, didx0, dsem0)
            s_start(rows0, didx0, ssem0)
            g_wait(rows1, sidx1, gsem1)
            d_wait(last + 1, didx1, dsem1)
            s_start(rows1, didx1, ssem1)
            s_wait(rows0, didx0, ssem0)
            s_wait(rows1, didx1, ssem1)
            plsc.subcore_barrier()

        @pl.when(cid == 0)
        def _():
            run(nct0, sid * nct0)

        @pl.when(cid == 1)
        def _():
            run(nct1, k0 + sid * nct1)

        pltpu.sync_copy(acc_sh.at[pl.ds(zrow, rows_per_tile)],
                        out_hbm.at[cid, pl.ds(zrow, rows_per_tile)])

    return seg_sum


def _stage_a_body(x_ref, w1_ref, dsw_ref, dsb_ref, s1_ref, ox_ref):
    x = x_ref[...]
    s1_ref[...] = jnp.dot(x, w1_ref[...], preferred_element_type=jnp.float32)
    ox_ref[...] = jnp.dot(x, dsw_ref[...],
                          preferred_element_type=jnp.float32) + dsb_ref[...]


def _stage_b_body(pa_ref, pb_ref, b1_ref, w2_ref, w3_ref, s2_ref, s3_ref):
    h = jnp.maximum(pa_ref[...] + pb_ref[...] + b1_ref[...], 0.0)
    s2_ref[...] = jnp.dot(h, w2_ref[...], preferred_element_type=jnp.float32)
    s3_ref[...] = jnp.dot(h, w3_ref[...], preferred_element_type=jnp.float32)


def _stage_c1_body(p3a_ref, p3b_ref, ox_ref, b3_ref,
                   l2w_ref, l2b_ref, l3w_ref, l3b_ref, r2_ref, r3_ref):
    h2 = jnp.maximum(p3a_ref[...] + p3b_ref[...] + b3_ref[...], 0.0)
    h2 = h2 + ox_ref[...]
    r2_ref[...] = jnp.dot(h2, l2w_ref[...],
                          preferred_element_type=jnp.float32) + l2b_ref[...]
    r3_ref[...] = jnp.dot(h2, l3w_ref[...],
                          preferred_element_type=jnp.float32) + l3b_ref[...]


def _stage_c2_body(p2a_ref, p2b_ref, b2_ref, r1_ref):
    nclass = b2_ref.shape[1]
    c = (p2a_ref[...] + p2b_ref[...])[:, :nclass] + b2_ref[...]
    c = c - jnp.max(c, axis=1, keepdims=True)
    r1_ref[...] = c - jnp.log(jnp.sum(jnp.exp(c), axis=1, keepdims=True))


_ROW_BLK = N_PAD // 8


def _row_spec(cols):
    return pl.BlockSpec((_ROW_BLK, cols), lambda i: (i, 0))


def _full_spec(rows, cols):
    return pl.BlockSpec((rows, cols), lambda i: (0, 0))


def kernel(x, adj, gc1_W, gc1_b, gc2_W, gc2_b, gc3_W, gc3_b, ds_W, ds_b,
           lin2_W, lin2_b, lin3_W, lin3_b):
    n, d = x.shape
    e = adj.shape[1]
    nclass = gc2_W.shape[1]
    ndeg = lin3_W.shape[1]
    w2pad = 48

    epw = NC * NS * CHUNK * 2
    e_pad = ((e + epw - 1) // epw) * epw
    pad = e_pad - e
    total_chunks_per_tile = e_pad // (NC * NS * CHUNK)
    nct0 = total_chunks_per_tile + 64
    nct1 = 2 * total_chunks_per_tile - nct0
    src = jnp.concatenate([adj[0], jnp.zeros((pad,), jnp.int32)])
    dst = jnp.concatenate([adj[1], jnp.full((pad,), n, jnp.int32)])
    src = src.reshape(e_pad // CHUNK, CHUNK)
    dst = dst.reshape(e_pad // CHUNK, CHUNK)
    x_p = jnp.concatenate([x, jnp.zeros((N_PAD - n, d), jnp.float32)])
    gc2_Wp = jnp.concatenate(
        [gc2_W, jnp.zeros((d, w2pad - nclass), jnp.float32)], axis=1)

    seg_sum_d = _seg_sum_kernel(nct0, nct1, d)
    seg_sum_n = _seg_sum_kernel(nct0, nct1, w2pad)

    s1, ox = pl.pallas_call(
        _stage_a_body,
        grid=(N_PAD // _ROW_BLK,),
        in_specs=[_row_spec(d), _full_spec(d, d), _full_spec(d, d),
                  _full_spec(1, d)],
        out_specs=[_row_spec(d), _row_spec(d)],
        out_shape=[jax.ShapeDtypeStruct((N_PAD, d), jnp.float32),
                   jax.ShapeDtypeStruct((N_PAD, d), jnp.float32)],
    )(x_p, gc1_W, ds_W, ds_b.reshape(1, d))

    p1 = seg_sum_d(s1, src, dst)

    s2, s3 = pl.pallas_call(
        _stage_b_body,
        grid=(N_PAD // _ROW_BLK,),
        in_specs=[_row_spec(d), _row_spec(d), _full_spec(1, d),
                  _full_spec(d, w2pad), _full_spec(d, d)],
        out_specs=[_row_spec(w2pad), _row_spec(d)],
        out_shape=[jax.ShapeDtypeStruct((N_PAD, w2pad), jnp.float32),
                   jax.ShapeDtypeStruct((N_PAD, d), jnp.float32)],
    )(p1[0], p1[1], gc1_b.reshape(1, d), gc2_Wp, gc3_W)

    p3 = seg_sum_d(s3, src, dst)
    p2 = seg_sum_n(s2, src, dst)

    r2, r3 = pl.pallas_call(
        _stage_c1_body,
        grid=(N_PAD // _ROW_BLK,),
        in_specs=[
            _row_spec(d), _row_spec(d), _row_spec(d), _full_spec(1, d),
            _full_spec(d, 1), _full_spec(1, 1),
            _full_spec(d, ndeg), _full_spec(1, ndeg),
        ],
        out_specs=[_row_spec(1), _row_spec(ndeg)],
        out_shape=[
            jax.ShapeDtypeStruct((N_PAD, 1), jnp.float32),
            jax.ShapeDtypeStruct((N_PAD, ndeg), jnp.float32),
        ],
    )(p3[0], p3[1], ox, gc3_b.reshape(1, d),
      lin2_W, lin2_b.reshape(1, 1),
      lin3_W, lin3_b.reshape(1, ndeg))

    r1 = pl.pallas_call(
        _stage_c2_body,
        grid=(N_PAD // _ROW_BLK,),
        in_specs=[_row_spec(w2pad), _row_spec(w2pad), _full_spec(1, nclass)],
        out_specs=_row_spec(nclass),
        out_shape=jax.ShapeDtypeStruct((N_PAD, nclass), jnp.float32),
    )(p2[0], p2[1], gc2_b.reshape(1, nclass))

    return (r1[:n], r2[:n, 0], r3[:n])
